# baseline (speedup 1.0000x reference)
import functools

import jax
import jax.numpy as jnp
from jax import lax
from jax.experimental import pallas as pl
from jax.experimental.pallas import tpu as pltpu
from jax.experimental.pallas import tpu_sc as plsc

B_ROWS = 4096
SEQ = 200
D_MODEL = 64
SCALE = float(D_MODEL) ** 0.5
LANES = 16

NC, NS = 2, 16
NW = NC * NS
BW = B_ROWS // NW
NBUF = 4
NOBUF = 2
ROUNDS = SEQ // NBUF
CT, CI = D_MODEL // 8, 8
BT = B_ROWS // 128


def _tec_body(xt_hbm, table_hbm, out_hbm, *sc):
    idx_v = sc[0]
    gbuf = sc[1:1 + NBUF]
    obuf = sc[1 + NBUF:1 + NBUF + NOBUF]
    gsem = sc[1 + NBUF + NOBUF:1 + 2 * NBUF + NOBUF]
    osem = sc[1 + 2 * NBUF + NOBUF:]

    wid = lax.axis_index("c") * NS + lax.axis_index("s")
    b0 = wid * BW

    pltpu.sync_copy(xt_hbm.at[:, pl.ds(b0, BW)], idx_v)

    def start_gather(b, t):
        pltpu.async_copy(table_hbm.at[idx_v.at[t]], gbuf[b], gsem[b])

    def wait_gather(b):
        pltpu.make_async_copy(table_hbm.at[idx_v.at[0]], gbuf[b], gsem[b]).wait()

    def start_out(ob, t):
        pltpu.async_copy(obuf[ob], out_hbm.at[t, :, wid], osem[ob])

    def wait_out(ob):
        pltpu.make_async_copy(obuf[ob], out_hbm.at[0, :, wid], osem[ob]).wait()

    row16 = [jax.lax.iota(jnp.int32, LANES) + (k * LANES) for k in range(BW // LANES)]

    def scale_t(b, ob):
        gb, o = gbuf[b], obuf[ob]

        def body_fn(c, carry):
            ct = c // CI
            ci = c % CI
            cvec = jnp.zeros((LANES,), jnp.int32) + c
            for k in range(BW // LANES):
                vals = plsc.load_gather(gb, [row16[k], cvec])
                o[ct, ci, pl.ds(k * LANES, LANES)] = vals * SCALE
            return carry

        lax.fori_loop(0, D_MODEL, body_fn, 0)

    for b in range(NBUF):
        start_gather(b, b)

    for b in range(NBUF):
        wait_gather(b)
        ob = b % NOBUF
        if b >= NOBUF:
            wait_out(ob)
        scale_t(b, ob)
        start_gather(b, b + NBUF)
        start_out(ob, b)

    def round_body(ro, carry):
        for b in range(NBUF):
            t = ro * NBUF + b
            wait_gather(b)
            ob = b % NOBUF
            wait_out(ob)
            scale_t(b, ob)
            start_gather(b, t + NBUF)
            start_out(ob, t)
        return carry

    lax.fori_loop(1, ROUNDS - 1, round_body, 0)

    for b in range(NBUF):
        t = (ROUNDS - 1) * NBUF + b
        wait_gather(b)
        ob = b % NOBUF
        wait_out(ob)
        scale_t(b, ob)
        start_out(ob, t)

    for ob in range(NOBUF):
        wait_out(ob)


_emb = functools.partial(
    pl.kernel,
    out_type=jax.ShapeDtypeStruct((SEQ, CT, BT, CI, 128), jnp.float32),
    mesh=plsc.VectorSubcoreMesh(core_axis_name="c", subcore_axis_name="s"),
    scratch_types=(
        [pltpu.VMEM((SEQ, BW), jnp.int32)]
        + [pltpu.VMEM((BW, D_MODEL), jnp.float32) for _ in range(NBUF)]
        + [pltpu.VMEM((CT, CI, 128), jnp.float32) for _ in range(NOBUF)]
        + [pltpu.SemaphoreType.DMA for _ in range(NBUF + NOBUF)]
    ),
    compiler_params=pltpu.CompilerParams(
        use_tc_tiling_on_sc=False, needs_layout_passes=False),
)(_tec_body)


def kernel(x, table):
    out5 = _emb(x.T.astype(jnp.int32), table)
    return jnp.transpose(out5, (2, 4, 0, 1, 3)).reshape(B_ROWS, SEQ, D_MODEL)


# speedup vs baseline: 1.4965x; 1.4965x over previous
import functools

import jax
import jax.numpy as jnp
from jax import lax
from jax.experimental import pallas as pl
from jax.experimental.pallas import tpu as pltpu
from jax.experimental.pallas import tpu_sc as plsc

B_ROWS = 4096
SEQ = 200
D_MODEL = 64
SCALE = float(D_MODEL) ** 0.5
LANES = 16

NC, NS = 2, 16
NW = NC * NS
BW = B_ROWS // NW
NBUF = 4
NOBUF = 2
ROUNDS = SEQ // NBUF
CT, CI = D_MODEL // 8, 8
BT = B_ROWS // 128
KG = BW // LANES


def _tec_body(xt_hbm, tpair_hbm, out_hbm, *sc):
    idx_v = sc[0]
    pidx = sc[1:1 + NBUF]
    gbuf = sc[1 + NBUF:1 + 2 * NBUF]
    obuf = sc[1 + 2 * NBUF:1 + 2 * NBUF + NOBUF]
    gsem = sc[1 + 2 * NBUF + NOBUF:1 + 3 * NBUF + NOBUF]
    osem = sc[1 + 3 * NBUF + NOBUF:]

    wid = lax.axis_index("c") * NS + lax.axis_index("s")
    b0 = wid * BW

    pltpu.sync_copy(xt_hbm.at[:, pl.ds(b0, BW)], idx_v)

    row16 = [jax.lax.iota(jnp.int32, LANES) + (k * LANES) for k in range(KG)]

    def start_gather(b, t):
        for k in range(KG):
            s = pl.ds(k * LANES, LANES)
            pidx[b][s] = jax.lax.shift_right_logical(idx_v[t, s], 1)
        pltpu.async_copy(tpair_hbm.at[pidx[b]], gbuf[b], gsem[b])

    def wait_gather(b):
        pltpu.make_async_copy(tpair_hbm.at[pidx[b]], gbuf[b], gsem[b]).wait()

    def start_out(ob, t):
        pltpu.async_copy(obuf[ob], out_hbm.at[t, :, wid], osem[ob])

    def wait_out(ob):
        pltpu.make_async_copy(obuf[ob], out_hbm.at[0, :, wid], osem[ob]).wait()

    def scale_t(b, ob, t):
        gb, o = gbuf[b], obuf[ob]
        hvec = [
            jax.lax.shift_left(
                jax.lax.bitwise_and(idx_v[t, pl.ds(k * LANES, LANES)], 1), 6)
            for k in range(KG)
        ]

        def body_fn(c):
            ct = c // CI
            ci = c % CI
            for k in range(KG):
                vals = plsc.load_gather(gb, [row16[k], hvec[k] + c])
                o[ct, ci, pl.ds(k * LANES, LANES)] = vals * SCALE

        plsc.parallel_loop(0, D_MODEL, 1, unroll=4)(body_fn)

    for b in range(NBUF):
        start_gather(b, b)

    for b in range(NBUF):
        wait_gather(b)
        ob = b % NOBUF
        if b >= NOBUF:
            wait_out(ob)
        scale_t(b, ob, b)
        start_gather(b, b + NBUF)
        start_out(ob, b)

    def round_body(ro, carry):
        for b in range(NBUF):
            t = ro * NBUF + b
            wait_gather(b)
            ob = b % NOBUF
            wait_out(ob)
            scale_t(b, ob, t)
            start_gather(b, t + NBUF)
            start_out(ob, t)
        return carry

    lax.fori_loop(1, ROUNDS - 1, round_body, 0)

    for b in range(NBUF):
        t = (ROUNDS - 1) * NBUF + b
        wait_gather(b)
        ob = b % NOBUF
        wait_out(ob)
        scale_t(b, ob, t)
        start_out(ob, t)

    for ob in range(NOBUF):
        wait_out(ob)


_emb = functools.partial(
    pl.kernel,
    out_type=jax.ShapeDtypeStruct((SEQ, CT, BT, CI, 128), jnp.float32),
    mesh=plsc.VectorSubcoreMesh(core_axis_name="c", subcore_axis_name="s"),
    scratch_types=(
        [pltpu.VMEM((SEQ, BW), jnp.int32)]
        + [pltpu.VMEM((BW,), jnp.int32) for _ in range(NBUF)]
        + [pltpu.VMEM((BW, 128), jnp.float32) for _ in range(NBUF)]
        + [pltpu.VMEM((CT, CI, 128), jnp.float32) for _ in range(NOBUF)]
        + [pltpu.SemaphoreType.DMA for _ in range(NBUF + NOBUF)]
    ),
    compiler_params=pltpu.CompilerParams(
        use_tc_tiling_on_sc=False, needs_layout_passes=False),
)(_tec_body)


def kernel(x, table):
    out5 = _emb(x.T.astype(jnp.int32), table.reshape(500000, 128))
    return jnp.transpose(out5, (2, 4, 0, 1, 3)).reshape(B_ROWS, SEQ, D_MODEL)
